# baseline (speedup 1.0000x reference)
import functools

import jax
import jax.numpy as jnp
from jax import lax
from jax.experimental import pallas as pl
from jax.experimental.pallas import tpu as pltpu
from jax.experimental.pallas import tpu_sc as plsc

EMB = 32
PACK = 128 // EMB
NC = 2
NS = 16
NW = NC * NS
IDX_CHUNK = 128


def _gather_sc(user_hi, item_hi, u_tab128, i_tab128, B):
    b_per_w = B // NW
    n_chunks = b_per_w // IDX_CHUNK
    mesh = plsc.VectorSubcoreMesh(core_axis_name="c", subcore_axis_name="s")

    @functools.partial(
        pl.kernel,
        mesh=mesh,
        out_type=(
            jax.ShapeDtypeStruct((B, 128), jnp.float32),
            jax.ShapeDtypeStruct((B, 128), jnp.float32),
        ),
        scratch_types=[
            pltpu.VMEM((n_chunks, IDX_CHUNK), jnp.int32),
            pltpu.VMEM((n_chunks, IDX_CHUNK), jnp.int32),
            pltpu.VMEM((2, IDX_CHUNK, 128), jnp.float32),
            pltpu.VMEM((2, IDX_CHUNK, 128), jnp.float32),
            pltpu.SemaphoreType.DMA,
        ],
    )
    def gather_kernel(u_idx_hbm, i_idx_hbm, u_tab, i_tab, u_out, i_out,
                      uidx_v, iidx_v, urows_v, irows_v, sem):
        wid = lax.axis_index("s") * NC + lax.axis_index("c")
        base = wid * b_per_w
        pltpu.sync_copy(u_idx_hbm.at[wid], uidx_v)
        pltpu.sync_copy(i_idx_hbm.at[wid], iidx_v)
        pending = [None, None]
        for j in range(n_chunks):
            p = j % 2
            pending[p] = (
                pltpu.async_copy(u_tab.at[uidx_v.at[j]], urows_v.at[p], sem),
                pltpu.async_copy(i_tab.at[iidx_v.at[j]], irows_v.at[p], sem),
            )
            if j > 0:
                q = (j - 1) % 2
                dst = pl.ds(base + (j - 1) * IDX_CHUNK, IDX_CHUNK)
                for c in pending[q]:
                    c.wait()
                pltpu.sync_copy(urows_v.at[q], u_out.at[dst])
                pltpu.sync_copy(irows_v.at[q], i_out.at[dst])
                pending[q] = None
        last = (n_chunks - 1) % 2
        dst = pl.ds(base + (n_chunks - 1) * IDX_CHUNK, IDX_CHUNK)
        for c in pending[last]:
            c.wait()
        pltpu.sync_copy(urows_v.at[last], u_out.at[dst])
        pltpu.sync_copy(irows_v.at[last], i_out.at[dst])

    return gather_kernel(user_hi, item_hi, u_tab128, i_tab128)


def _select_segment(rows, sub):
    seg = jnp.zeros((rows.shape[0], EMB), jnp.float32)
    for k in range(PACK):
        part = rows[:, k * EMB:(k + 1) * EMB]
        seg = seg + jnp.where(sub == k, part, 0.0)
    return seg


def _mlp_body(u_ref, i_ref, usub_ref, isub_ref, w1u_ref, w1i_ref, b1_ref,
              w2_ref, b2_ref, w3_ref, b3_ref, o_ref):
    u = _select_segment(u_ref[...], usub_ref[...])
    i = _select_segment(i_ref[...], isub_ref[...])
    x1 = jnp.dot(u, w1u_ref[...], preferred_element_type=jnp.float32)
    x1 += jnp.dot(i, w1i_ref[...], preferred_element_type=jnp.float32)
    x1 = jnp.maximum(x1 + b1_ref[...], 0.0)
    x2 = jnp.dot(x1, w2_ref[...], preferred_element_type=jnp.float32)
    x2 = jnp.maximum(x2 + b2_ref[...], 0.0)
    logits = jnp.dot(x2, w3_ref[...], preferred_element_type=jnp.float32)
    logits = logits + b3_ref[...]
    m = jnp.max(logits, axis=1, keepdims=True)
    e = jnp.exp(logits - m)
    o_ref[...] = e / jnp.sum(e, axis=1, keepdims=True)


def _mlp_tc(u_rows, i_rows, usub, isub, W1uT, W1iT, b1, W2T, b2, W3T, b3,
            interpret=False):
    B = u_rows.shape[0]
    BLK = 2048
    n_out = W3T.shape[1]
    full = lambda shape: pl.BlockSpec(shape, lambda i: (0, 0))
    return pl.pallas_call(
        _mlp_body,
        grid=(B // BLK,),
        in_specs=[
            pl.BlockSpec((BLK, 128), lambda i: (i, 0)),
            pl.BlockSpec((BLK, 128), lambda i: (i, 0)),
            pl.BlockSpec((BLK, 1), lambda i: (i, 0)),
            pl.BlockSpec((BLK, 1), lambda i: (i, 0)),
            full(W1uT.shape),
            full(W1iT.shape),
            full(b1.shape),
            full(W2T.shape),
            full(b2.shape),
            full(W3T.shape),
            full(b3.shape),
        ],
        out_specs=pl.BlockSpec((BLK, n_out), lambda i: (i, 0)),
        out_shape=jax.ShapeDtypeStruct((B, n_out), jnp.float32),
        interpret=interpret,
    )(u_rows, i_rows, usub, isub, W1uT, W1iT, b1, W2T, b2, W3T, b3)


def kernel(user, item, user_table, item_table, W1, b1, W2, b2, W3, b3):
    B = user.shape[0]
    b_per_w = B // NW
    n_chunks = b_per_w // IDX_CHUNK
    user = user.astype(jnp.int32)
    item = item.astype(jnp.int32)
    user_hi = (user // PACK).reshape(NW, n_chunks, IDX_CHUNK)
    item_hi = (item // PACK).reshape(NW, n_chunks, IDX_CHUNK)
    usub = (user % PACK).reshape(B, 1)
    isub = (item % PACK).reshape(B, 1)
    V = user_table.shape[0]
    u_tab128 = user_table.reshape(V // PACK, 128)
    i_tab128 = item_table.reshape(V // PACK, 128)
    u_rows, i_rows = _gather_sc(user_hi, item_hi, u_tab128, i_tab128, B)
    W1uT = W1[:, :EMB].T
    W1iT = W1[:, EMB:].T
    return _mlp_tc(u_rows, i_rows, usub, isub, W1uT, W1iT, b1.reshape(1, -1),
                   W2.T, b2.reshape(1, -1), W3.T, b3.reshape(1, -1))
